# baseline (speedup 1.0000x reference)
import functools

import jax
import jax.numpy as jnp
from jax import lax
from jax.experimental import pallas as pl
from jax.experimental.pallas import tpu as pltpu
from jax.experimental.pallas import tpu_sc as plsc

N = 10000
E = 320000
D = 128

NUM_CORES = 2
NUM_SUBCORES = 16
NUM_WORKERS = NUM_CORES * NUM_SUBCORES
ROW_CHUNK = 80
NUM_ROW_CHUNKS = N // ROW_CHUNK
ROW_CHUNKS_PER_SUBCORE = -(-NUM_ROW_CHUNKS // NUM_SUBCORES)

CHUNK = 128
WCHUNKS = 80
EDGES_PER_WORKER = WCHUNKS * CHUNK
EDGES_PER_CORE = EDGES_PER_WORKER * NUM_SUBCORES
E_PAD = EDGES_PER_CORE * NUM_CORES
ACC_ROWS = N + 128

DEG_W = 128


def _sc_mesh():
  return plsc.VectorSubcoreMesh(core_axis_name="c", subcore_axis_name="s")


def _sc_degree(dst):
  @functools.partial(
      pl.kernel,
      mesh=_sc_mesh(),
      out_type=jax.ShapeDtypeStruct((NUM_CORES, N, DEG_W), jnp.float32),
      scratch_types=[
          pltpu.VMEM((CHUNK,), jnp.int32),
          pltpu.VMEM((CHUNK,), jnp.int32),
          pltpu.VMEM((CHUNK, DEG_W), jnp.float32),
          pltpu.VMEM((ROW_CHUNK, DEG_W), jnp.float32),
          pltpu.VMEM_SHARED((ACC_ROWS, DEG_W), jnp.float32),
          pltpu.SemaphoreType.DMA,
          pltpu.SemaphoreType.DMA,
      ],
  )
  def deg_kernel(dst_hbm, out_hbm, idx_d0, idx_d1, ones_v, zbuf, acc,
                 ssem0, ssem1):
    cid = lax.axis_index("c")
    sid = lax.axis_index("s")

    @pl.loop(0, CHUNK)
    def _(r):
      @pl.loop(0, DEG_W, step=16)
      def _(j):
        ones_v[r, pl.ds(j, 16)] = jnp.ones((16,), jnp.float32)

    @pl.loop(0, ROW_CHUNK)
    def _(r):
      @pl.loop(0, DEG_W, step=16)
      def _(j):
        zbuf[r, pl.ds(j, 16)] = jnp.zeros((16,), jnp.float32)

    @pl.loop(0, ROW_CHUNKS_PER_SUBCORE)
    def _(z):
      c = z * NUM_SUBCORES + sid

      @pl.when(c < NUM_ROW_CHUNKS)
      def _():
        pltpu.sync_copy(zbuf, acc.at[pl.ds(c * ROW_CHUNK, ROW_CHUNK)])

    plsc.subcore_barrier()

    base0 = cid * EDGES_PER_CORE + sid * EDGES_PER_WORKER

    def deg_half(k, idx_d, ssem):
      @pl.when(k >= 2)
      def _():
        pltpu.make_async_copy(ones_v, acc.at[idx_d], ssem).wait()
      pltpu.sync_copy(dst_hbm.at[pl.ds(base0 + k * CHUNK, CHUNK)], idx_d)
      pltpu.async_copy(ones_v, acc.at[idx_d], ssem, add=True)

    @pl.loop(0, WCHUNKS, step=2)
    def _(k):
      deg_half(k, idx_d0, ssem0)
      deg_half(k + 1, idx_d1, ssem1)

    pltpu.make_async_copy(ones_v, acc.at[idx_d0], ssem0).wait()
    pltpu.make_async_copy(ones_v, acc.at[idx_d1], ssem1).wait()

    plsc.subcore_barrier()

    @pl.loop(0, ROW_CHUNKS_PER_SUBCORE)
    def _(z):
      c = z * NUM_SUBCORES + sid

      @pl.when(c < NUM_ROW_CHUNKS)
      def _():
        pltpu.sync_copy(
            acc.at[pl.ds(c * ROW_CHUNK, ROW_CHUNK)],
            out_hbm.at[cid].at[pl.ds(c * ROW_CHUNK, ROW_CHUNK)],
        )

  return deg_kernel(dst)


def _sc_aggregate(rows_hbm, src, dst):
  @functools.partial(
      pl.kernel,
      mesh=_sc_mesh(),
      out_type=jax.ShapeDtypeStruct((NUM_CORES, N, D), jnp.float32),
      scratch_types=[
          pltpu.VMEM((CHUNK,), jnp.int32),
          pltpu.VMEM((CHUNK,), jnp.int32),
          pltpu.VMEM((CHUNK,), jnp.int32),
          pltpu.VMEM((CHUNK,), jnp.int32),
          pltpu.VMEM((CHUNK, D), jnp.float32),
          pltpu.VMEM((CHUNK, D), jnp.float32),
          pltpu.VMEM((ROW_CHUNK, D), jnp.float32),
          pltpu.VMEM_SHARED((ACC_ROWS, D), jnp.float32),
          pltpu.SemaphoreType.DMA,
          pltpu.SemaphoreType.DMA,
          pltpu.SemaphoreType.DMA,
          pltpu.SemaphoreType.DMA,
      ],
  )
  def agg_kernel(rows_ref, src_hbm, dst_hbm, out_hbm,
                 idx_s0, idx_s1, idx_d0, idx_d1, rows0, rows1, zbuf, acc,
                 gsem0, gsem1, ssem0, ssem1):
    cid = lax.axis_index("c")
    sid = lax.axis_index("s")

    @pl.loop(0, ROW_CHUNK)
    def _(r):
      @pl.loop(0, D, step=16)
      def _(j):
        zbuf[r, pl.ds(j, 16)] = jnp.zeros((16,), jnp.float32)

    @pl.loop(0, ROW_CHUNKS_PER_SUBCORE)
    def _(z):
      c = z * NUM_SUBCORES + sid

      @pl.when(c < NUM_ROW_CHUNKS)
      def _():
        pltpu.sync_copy(zbuf, acc.at[pl.ds(c * ROW_CHUNK, ROW_CHUNK)])

    plsc.subcore_barrier()

    base0 = cid * EDGES_PER_CORE + sid * EDGES_PER_WORKER
    parity = ((idx_s0, idx_d0, rows0, gsem0, ssem0),
              (idx_s1, idx_d1, rows1, gsem1, ssem1))

    def agg_half(k, p):
      idx_s, idx_d, rows_v, gsem, ssem = parity[p]
      idx_sq, idx_dq, rows_q, gsem_q, ssem_q = parity[1 - p]

      @pl.when(k + 1 < WCHUNKS)
      def _():
        @pl.when(k >= 1)
        def _():
          pltpu.make_async_copy(rows_q, acc.at[idx_dq], ssem_q).wait()
        base = base0 + (k + 1) * CHUNK
        pltpu.sync_copy(src_hbm.at[pl.ds(base, CHUNK)], idx_sq)
        pltpu.sync_copy(dst_hbm.at[pl.ds(base, CHUNK)], idx_dq)
        pltpu.async_copy(rows_ref.at[idx_sq], rows_q, gsem_q)

      pltpu.make_async_copy(rows_ref.at[idx_s], rows_v, gsem).wait()
      pltpu.async_copy(rows_v, acc.at[idx_d], ssem, add=True)

    pltpu.sync_copy(src_hbm.at[pl.ds(base0, CHUNK)], idx_s0)
    pltpu.sync_copy(dst_hbm.at[pl.ds(base0, CHUNK)], idx_d0)
    pltpu.async_copy(rows_ref.at[idx_s0], rows0, gsem0)

    @pl.loop(0, WCHUNKS, step=2)
    def _(k):
      agg_half(k, 0)
      agg_half(k + 1, 1)

    pltpu.make_async_copy(rows0, acc.at[idx_d0], ssem0).wait()
    pltpu.make_async_copy(rows1, acc.at[idx_d1], ssem1).wait()

    plsc.subcore_barrier()

    @pl.loop(0, ROW_CHUNKS_PER_SUBCORE)
    def _(z):
      c = z * NUM_SUBCORES + sid

      @pl.when(c < NUM_ROW_CHUNKS)
      def _():
        pltpu.sync_copy(
            acc.at[pl.ds(c * ROW_CHUNK, ROW_CHUNK)],
            out_hbm.at[cid].at[pl.ds(c * ROW_CHUNK, ROW_CHUNK)],
        )

  return agg_kernel(rows_hbm, src, dst)


_BLK = 1000
_GRID = N // _BLK


def _tc_matmul(x, w):
  def body(x_ref, w_ref, o_ref):
    o_ref[...] = jnp.dot(x_ref[...], w_ref[...],
                         preferred_element_type=jnp.float32)

  return pl.pallas_call(
      body,
      grid=(_GRID,),
      in_specs=[
          pl.BlockSpec((_BLK, D), lambda i: (i, 0)),
          pl.BlockSpec((D, D), lambda i: (0, 0)),
      ],
      out_specs=pl.BlockSpec((_BLK, D), lambda i: (i, 0)),
      out_shape=jax.ShapeDtypeStruct((N, D), jnp.float32),
  )(x, w)


def _dinv_from_deg(degp_ref):
  deg = degp_ref[0, :, 0:1] + degp_ref[1, :, 0:1] + 1.0
  return lax.rsqrt(deg)


def _tc_scale(h, degp):
  def body(h_ref, degp_ref, o_ref):
    o_ref[...] = h_ref[...] * _dinv_from_deg(degp_ref)

  return pl.pallas_call(
      body,
      grid=(_GRID,),
      in_specs=[
          pl.BlockSpec((_BLK, D), lambda i: (i, 0)),
          pl.BlockSpec((NUM_CORES, _BLK, DEG_W), lambda i: (0, i, 0)),
      ],
      out_specs=pl.BlockSpec((_BLK, D), lambda i: (i, 0)),
      out_shape=jax.ShapeDtypeStruct((N, D), jnp.float32),
  )(h, degp)


def _tc_epilogue_mm(accp, hs, degp, b, w):
  def body(accp_ref, hs_ref, degp_ref, b_ref, w_ref, o_ref):
    dinv = _dinv_from_deg(degp_ref)
    t = dinv * (accp_ref[0] + accp_ref[1] + hs_ref[...]) + b_ref[...]
    t = jnp.maximum(t, 0.0)
    o_ref[...] = jnp.dot(t, w_ref[...],
                         preferred_element_type=jnp.float32) * dinv

  return pl.pallas_call(
      body,
      grid=(_GRID,),
      in_specs=[
          pl.BlockSpec((NUM_CORES, _BLK, D), lambda i: (0, i, 0)),
          pl.BlockSpec((_BLK, D), lambda i: (i, 0)),
          pl.BlockSpec((NUM_CORES, _BLK, DEG_W), lambda i: (0, i, 0)),
          pl.BlockSpec((1, D), lambda i: (0, 0)),
          pl.BlockSpec((D, D), lambda i: (0, 0)),
      ],
      out_specs=pl.BlockSpec((_BLK, D), lambda i: (i, 0)),
      out_shape=jax.ShapeDtypeStruct((N, D), jnp.float32),
  )(accp, hs, degp, b, w)


def _tc_epilogue(accp, hs, degp, b):
  def body(accp_ref, hs_ref, degp_ref, b_ref, o_ref):
    dinv = _dinv_from_deg(degp_ref)
    t = dinv * (accp_ref[0] + accp_ref[1] + hs_ref[...]) + b_ref[...]
    o_ref[...] = jnp.maximum(t, 0.0)

  return pl.pallas_call(
      body,
      grid=(_GRID,),
      in_specs=[
          pl.BlockSpec((NUM_CORES, _BLK, D), lambda i: (0, i, 0)),
          pl.BlockSpec((_BLK, D), lambda i: (i, 0)),
          pl.BlockSpec((NUM_CORES, _BLK, DEG_W), lambda i: (0, i, 0)),
          pl.BlockSpec((1, D), lambda i: (0, 0)),
      ],
      out_specs=pl.BlockSpec((_BLK, D), lambda i: (i, 0)),
      out_shape=jax.ShapeDtypeStruct((N, D), jnp.float32),
  )(accp, hs, degp, b)


def kernel(x, edge_index, W1, b1, W2, b2):
  npad = E_PAD - E
  pad_dst = N + (jnp.arange(npad, dtype=jnp.int32) % (ACC_ROWS - N))
  src = jnp.concatenate(
      [edge_index[0].astype(jnp.int32), jnp.zeros((npad,), jnp.int32)])
  dst = jnp.concatenate([edge_index[1].astype(jnp.int32), pad_dst])
  b1r = b1.reshape(1, D)
  b2r = b2.reshape(1, D)

  degp = _sc_degree(dst)
  h1 = _tc_matmul(x, W1)

  hs1 = _tc_scale(h1, degp)
  acc1 = _sc_aggregate(hs1, src, dst)
  hs2 = _tc_epilogue_mm(acc1, hs1, degp, b1r, W2)
  acc2 = _sc_aggregate(hs2, src, dst)
  return _tc_epilogue(acc2, hs2, degp, b2r)


# speedup vs baseline: 1.2445x; 1.2445x over previous
import functools

import jax
import jax.numpy as jnp
from jax import lax
from jax.experimental import pallas as pl
from jax.experimental.pallas import tpu as pltpu
from jax.experimental.pallas import tpu_sc as plsc

N = 10000
E = 320000
D = 128

NUM_CORES = 2
NUM_SUBCORES = 16
ROW_CHUNK = 80
NUM_ROW_CHUNKS = N // ROW_CHUNK
ROW_CHUNKS_PER_SUBCORE = -(-NUM_ROW_CHUNKS // NUM_SUBCORES)
EDGES_PER_CORE = E // NUM_CORES
EDGES_PER_WORKER = EDGES_PER_CORE // NUM_SUBCORES
CHUNK = 80
CHUNKS_PER_WORKER = EDGES_PER_WORKER // CHUNK

DEG_W = 128


def _sc_mesh():
  return plsc.VectorSubcoreMesh(core_axis_name="c", subcore_axis_name="s")


def _sc_degree(dst):
  @functools.partial(
      pl.kernel,
      mesh=_sc_mesh(),
      out_type=jax.ShapeDtypeStruct((NUM_CORES, N, DEG_W), jnp.float32),
      scratch_types=[
          pltpu.VMEM((CHUNK,), jnp.int32),
          pltpu.VMEM((CHUNK, DEG_W), jnp.float32),
          pltpu.VMEM((ROW_CHUNK, DEG_W), jnp.float32),
          pltpu.VMEM_SHARED((N, DEG_W), jnp.float32),
          pltpu.SemaphoreType.DMA,
      ],
  )
  def deg_kernel(dst_hbm, out_hbm, idx_d, ones_v, zbuf, acc, sem):
    cid = lax.axis_index("c")
    sid = lax.axis_index("s")

    @pl.loop(0, CHUNK)
    def _(r):
      @pl.loop(0, DEG_W, step=16)
      def _(j):
        ones_v[r, pl.ds(j, 16)] = jnp.ones((16,), jnp.float32)

    @pl.loop(0, ROW_CHUNK)
    def _(r):
      @pl.loop(0, DEG_W, step=16)
      def _(j):
        zbuf[r, pl.ds(j, 16)] = jnp.zeros((16,), jnp.float32)

    @pl.loop(0, ROW_CHUNKS_PER_SUBCORE)
    def _(z):
      c = z * NUM_SUBCORES + sid

      @pl.when(c < NUM_ROW_CHUNKS)
      def _():
        pltpu.sync_copy(zbuf, acc.at[pl.ds(c * ROW_CHUNK, ROW_CHUNK)])

    plsc.subcore_barrier()

    base0 = cid * EDGES_PER_CORE + sid * EDGES_PER_WORKER

    @pl.loop(0, CHUNKS_PER_WORKER)
    def _(i):
      pltpu.sync_copy(dst_hbm.at[pl.ds(base0 + i * CHUNK, CHUNK)], idx_d)
      pltpu.sync_copy(ones_v, acc.at[idx_d], add=True)

    plsc.subcore_barrier()

    @pl.loop(0, ROW_CHUNKS_PER_SUBCORE)
    def _(z):
      c = z * NUM_SUBCORES + sid

      @pl.when(c < NUM_ROW_CHUNKS)
      def _():
        pltpu.sync_copy(
            acc.at[pl.ds(c * ROW_CHUNK, ROW_CHUNK)],
            out_hbm.at[cid].at[pl.ds(c * ROW_CHUNK, ROW_CHUNK)],
        )

  return deg_kernel(dst)


def _sc_aggregate(rows_hbm, src, dst):
  @functools.partial(
      pl.kernel,
      mesh=_sc_mesh(),
      out_type=jax.ShapeDtypeStruct((NUM_CORES, N, D), jnp.float32),
      scratch_types=[
          pltpu.VMEM((CHUNK,), jnp.int32),
          pltpu.VMEM((CHUNK,), jnp.int32),
          pltpu.VMEM((CHUNK, D), jnp.float32),
          pltpu.VMEM((ROW_CHUNK, D), jnp.float32),
          pltpu.VMEM_SHARED((N, D), jnp.float32),
          pltpu.SemaphoreType.DMA,
      ],
  )
  def agg_kernel(rows_ref, src_hbm, dst_hbm, out_hbm,
                 idx_s, idx_d, rows_v, zbuf, acc, sem):
    cid = lax.axis_index("c")
    sid = lax.axis_index("s")

    @pl.loop(0, ROW_CHUNK)
    def _(r):
      @pl.loop(0, D, step=16)
      def _(j):
        zbuf[r, pl.ds(j, 16)] = jnp.zeros((16,), jnp.float32)

    @pl.loop(0, ROW_CHUNKS_PER_SUBCORE)
    def _(z):
      c = z * NUM_SUBCORES + sid

      @pl.when(c < NUM_ROW_CHUNKS)
      def _():
        pltpu.sync_copy(zbuf, acc.at[pl.ds(c * ROW_CHUNK, ROW_CHUNK)])

    plsc.subcore_barrier()

    base0 = cid * EDGES_PER_CORE + sid * EDGES_PER_WORKER

    @pl.loop(0, CHUNKS_PER_WORKER)
    def _(i):
      base = base0 + i * CHUNK
      pltpu.sync_copy(src_hbm.at[pl.ds(base, CHUNK)], idx_s)
      pltpu.sync_copy(dst_hbm.at[pl.ds(base, CHUNK)], idx_d)
      pltpu.async_copy(rows_ref.at[idx_s], rows_v, sem).wait()
      pltpu.sync_copy(rows_v, acc.at[idx_d], add=True)

    plsc.subcore_barrier()

    @pl.loop(0, ROW_CHUNKS_PER_SUBCORE)
    def _(z):
      c = z * NUM_SUBCORES + sid

      @pl.when(c < NUM_ROW_CHUNKS)
      def _():
        pltpu.sync_copy(
            acc.at[pl.ds(c * ROW_CHUNK, ROW_CHUNK)],
            out_hbm.at[cid].at[pl.ds(c * ROW_CHUNK, ROW_CHUNK)],
        )

  return agg_kernel(rows_hbm, src, dst)


_BLK = 1000
_GRID = N // _BLK


def _tc_matmul(x, w):
  def body(x_ref, w_ref, o_ref):
    o_ref[...] = jnp.dot(x_ref[...], w_ref[...],
                         preferred_element_type=jnp.float32)

  return pl.pallas_call(
      body,
      grid=(_GRID,),
      in_specs=[
          pl.BlockSpec((_BLK, D), lambda i: (i, 0)),
          pl.BlockSpec((D, D), lambda i: (0, 0)),
      ],
      out_specs=pl.BlockSpec((_BLK, D), lambda i: (i, 0)),
      out_shape=jax.ShapeDtypeStruct((N, D), jnp.float32),
  )(x, w)


def _dinv_from_deg(degp_ref):
  deg = degp_ref[0, :, 0:1] + degp_ref[1, :, 0:1] + 1.0
  return lax.rsqrt(deg)


def _tc_scale(h, degp):
  def body(h_ref, degp_ref, o_ref):
    o_ref[...] = h_ref[...] * _dinv_from_deg(degp_ref)

  return pl.pallas_call(
      body,
      grid=(_GRID,),
      in_specs=[
          pl.BlockSpec((_BLK, D), lambda i: (i, 0)),
          pl.BlockSpec((NUM_CORES, _BLK, DEG_W), lambda i: (0, i, 0)),
      ],
      out_specs=pl.BlockSpec((_BLK, D), lambda i: (i, 0)),
      out_shape=jax.ShapeDtypeStruct((N, D), jnp.float32),
  )(h, degp)


def _tc_epilogue_mm(accp, hs, degp, b, w):
  def body(accp_ref, hs_ref, degp_ref, b_ref, w_ref, o_ref):
    dinv = _dinv_from_deg(degp_ref)
    t = dinv * (accp_ref[0] + accp_ref[1] + hs_ref[...]) + b_ref[...]
    t = jnp.maximum(t, 0.0)
    o_ref[...] = jnp.dot(t, w_ref[...],
                         preferred_element_type=jnp.float32) * dinv

  return pl.pallas_call(
      body,
      grid=(_GRID,),
      in_specs=[
          pl.BlockSpec((NUM_CORES, _BLK, D), lambda i: (0, i, 0)),
          pl.BlockSpec((_BLK, D), lambda i: (i, 0)),
          pl.BlockSpec((NUM_CORES, _BLK, DEG_W), lambda i: (0, i, 0)),
          pl.BlockSpec((1, D), lambda i: (0, 0)),
          pl.BlockSpec((D, D), lambda i: (0, 0)),
      ],
      out_specs=pl.BlockSpec((_BLK, D), lambda i: (i, 0)),
      out_shape=jax.ShapeDtypeStruct((N, D), jnp.float32),
  )(accp, hs, degp, b, w)


def _tc_epilogue(accp, hs, degp, b):
  def body(accp_ref, hs_ref, degp_ref, b_ref, o_ref):
    dinv = _dinv_from_deg(degp_ref)
    t = dinv * (accp_ref[0] + accp_ref[1] + hs_ref[...]) + b_ref[...]
    o_ref[...] = jnp.maximum(t, 0.0)

  return pl.pallas_call(
      body,
      grid=(_GRID,),
      in_specs=[
          pl.BlockSpec((NUM_CORES, _BLK, D), lambda i: (0, i, 0)),
          pl.BlockSpec((_BLK, D), lambda i: (i, 0)),
          pl.BlockSpec((NUM_CORES, _BLK, DEG_W), lambda i: (0, i, 0)),
          pl.BlockSpec((1, D), lambda i: (0, 0)),
      ],
      out_specs=pl.BlockSpec((_BLK, D), lambda i: (i, 0)),
      out_shape=jax.ShapeDtypeStruct((N, D), jnp.float32),
  )(accp, hs, degp, b)


def kernel(x, edge_index, W1, b1, W2, b2):
  src = edge_index[0].astype(jnp.int32)
  dst = edge_index[1].astype(jnp.int32)
  b1r = b1.reshape(1, D)
  b2r = b2.reshape(1, D)

  degp = _sc_degree(dst)
  h1 = _tc_matmul(x, W1)

  hs1 = _tc_scale(h1, degp)
  acc1 = _sc_aggregate(hs1, src, dst)
  hs2 = _tc_epilogue_mm(acc1, hs1, degp, b1r, W2)
  acc2 = _sc_aggregate(hs2, src, dst)
  return _tc_epilogue(acc2, hs2, degp, b2r)


# speedup vs baseline: 2.0555x; 1.6517x over previous
import functools

import jax
import jax.numpy as jnp
from jax import lax
from jax.experimental import pallas as pl
from jax.experimental.pallas import tpu as pltpu
from jax.experimental.pallas import tpu_sc as plsc

N = 10000
E = 320000
D = 128

NUM_CORES = 2
NUM_SUBCORES = 16
ROW_CHUNK = 80
NUM_ROW_CHUNKS = N // ROW_CHUNK
ROW_CHUNKS_PER_SUBCORE = -(-NUM_ROW_CHUNKS // NUM_SUBCORES)
EDGES_PER_CORE = E // NUM_CORES
EDGES_PER_WORKER = EDGES_PER_CORE // NUM_SUBCORES
CHUNK = 125
CHUNKS_PER_WORKER = EDGES_PER_WORKER // CHUNK
IDX_ROWS = E // CHUNK
IDX_ROWS_PER_CORE = IDX_ROWS // NUM_CORES

DEG_W = 128


def _sc_mesh():
  return plsc.VectorSubcoreMesh(core_axis_name="c", subcore_axis_name="s")


def _sc_degree(dst):
  @functools.partial(
      pl.kernel,
      mesh=_sc_mesh(),
      out_type=jax.ShapeDtypeStruct((NUM_CORES, N, DEG_W), jnp.float32),
      scratch_types=[
          pltpu.VMEM((CHUNKS_PER_WORKER, CHUNK), jnp.int32),
          pltpu.VMEM((CHUNK, DEG_W), jnp.float32),
          pltpu.VMEM((ROW_CHUNK, DEG_W), jnp.float32),
          pltpu.VMEM_SHARED((N, DEG_W), jnp.float32),
          pltpu.SemaphoreType.DMA,
      ],
  )
  def deg_kernel(dst_hbm, out_hbm, idx_d, ones_v, zbuf, acc, sem):
    cid = lax.axis_index("c")
    sid = lax.axis_index("s")

    irow = cid * IDX_ROWS_PER_CORE + sid * CHUNKS_PER_WORKER
    pltpu.sync_copy(dst_hbm.at[pl.ds(irow, CHUNKS_PER_WORKER)], idx_d)

    @pl.loop(0, CHUNK)
    def _(r):
      @pl.loop(0, DEG_W, step=16)
      def _(j):
        ones_v[r, pl.ds(j, 16)] = jnp.ones((16,), jnp.float32)

    @pl.loop(0, ROW_CHUNK)
    def _(r):
      @pl.loop(0, DEG_W, step=16)
      def _(j):
        zbuf[r, pl.ds(j, 16)] = jnp.zeros((16,), jnp.float32)

    @pl.loop(0, ROW_CHUNKS_PER_SUBCORE)
    def _(z):
      c = z * NUM_SUBCORES + sid

      @pl.when(c < NUM_ROW_CHUNKS)
      def _():
        pltpu.sync_copy(zbuf, acc.at[pl.ds(c * ROW_CHUNK, ROW_CHUNK)])

    plsc.subcore_barrier()

    @pl.loop(0, CHUNKS_PER_WORKER)
    def _(i):
      pltpu.sync_copy(ones_v, acc.at[idx_d.at[i]], add=True)

    plsc.subcore_barrier()

    @pl.loop(0, ROW_CHUNKS_PER_SUBCORE)
    def _(z):
      c = z * NUM_SUBCORES + sid

      @pl.when(c < NUM_ROW_CHUNKS)
      def _():
        pltpu.sync_copy(
            acc.at[pl.ds(c * ROW_CHUNK, ROW_CHUNK)],
            out_hbm.at[cid].at[pl.ds(c * ROW_CHUNK, ROW_CHUNK)],
        )

  return deg_kernel(dst)


def _sc_aggregate(rows_hbm, src, dst):
  @functools.partial(
      pl.kernel,
      mesh=_sc_mesh(),
      out_type=jax.ShapeDtypeStruct((NUM_CORES, N, D), jnp.float32),
      scratch_types=[
          pltpu.VMEM((CHUNKS_PER_WORKER, CHUNK), jnp.int32),
          pltpu.VMEM((CHUNKS_PER_WORKER, CHUNK), jnp.int32),
          pltpu.VMEM((CHUNK, D), jnp.float32),
          pltpu.VMEM((ROW_CHUNK, D), jnp.float32),
          pltpu.VMEM_SHARED((N, D), jnp.float32),
          pltpu.SemaphoreType.DMA,
      ],
  )
  def agg_kernel(rows_ref, src_hbm, dst_hbm, out_hbm,
                 idx_s, idx_d, rows_v, zbuf, acc, sem):
    cid = lax.axis_index("c")
    sid = lax.axis_index("s")

    irow = cid * IDX_ROWS_PER_CORE + sid * CHUNKS_PER_WORKER
    pltpu.sync_copy(src_hbm.at[pl.ds(irow, CHUNKS_PER_WORKER)], idx_s)
    pltpu.sync_copy(dst_hbm.at[pl.ds(irow, CHUNKS_PER_WORKER)], idx_d)

    @pl.loop(0, ROW_CHUNK)
    def _(r):
      @pl.loop(0, D, step=16)
      def _(j):
        zbuf[r, pl.ds(j, 16)] = jnp.zeros((16,), jnp.float32)

    @pl.loop(0, ROW_CHUNKS_PER_SUBCORE)
    def _(z):
      c = z * NUM_SUBCORES + sid

      @pl.when(c < NUM_ROW_CHUNKS)
      def _():
        pltpu.sync_copy(zbuf, acc.at[pl.ds(c * ROW_CHUNK, ROW_CHUNK)])

    plsc.subcore_barrier()

    @pl.loop(0, CHUNKS_PER_WORKER)
    def _(i):
      pltpu.async_copy(rows_ref.at[idx_s.at[i]], rows_v, sem).wait()
      pltpu.sync_copy(rows_v, acc.at[idx_d.at[i]], add=True)

    plsc.subcore_barrier()

    @pl.loop(0, ROW_CHUNKS_PER_SUBCORE)
    def _(z):
      c = z * NUM_SUBCORES + sid

      @pl.when(c < NUM_ROW_CHUNKS)
      def _():
        pltpu.sync_copy(
            acc.at[pl.ds(c * ROW_CHUNK, ROW_CHUNK)],
            out_hbm.at[cid].at[pl.ds(c * ROW_CHUNK, ROW_CHUNK)],
        )

  return agg_kernel(rows_hbm, src, dst)


_BLK = 1000
_GRID = N // _BLK


def _tc_matmul(x, w):
  def body(x_ref, w_ref, o_ref):
    o_ref[...] = jnp.dot(x_ref[...], w_ref[...],
                         preferred_element_type=jnp.float32)

  return pl.pallas_call(
      body,
      grid=(_GRID,),
      in_specs=[
          pl.BlockSpec((_BLK, D), lambda i: (i, 0)),
          pl.BlockSpec((D, D), lambda i: (0, 0)),
      ],
      out_specs=pl.BlockSpec((_BLK, D), lambda i: (i, 0)),
      out_shape=jax.ShapeDtypeStruct((N, D), jnp.float32),
  )(x, w)


def _dinv_from_deg(degp_ref):
  deg = degp_ref[0, :, 0:1] + degp_ref[1, :, 0:1] + 1.0
  return lax.rsqrt(deg)


def _tc_scale(h, degp):
  def body(h_ref, degp_ref, o_ref):
    o_ref[...] = h_ref[...] * _dinv_from_deg(degp_ref)

  return pl.pallas_call(
      body,
      grid=(_GRID,),
      in_specs=[
          pl.BlockSpec((_BLK, D), lambda i: (i, 0)),
          pl.BlockSpec((NUM_CORES, _BLK, DEG_W), lambda i: (0, i, 0)),
      ],
      out_specs=pl.BlockSpec((_BLK, D), lambda i: (i, 0)),
      out_shape=jax.ShapeDtypeStruct((N, D), jnp.float32),
  )(h, degp)


def _tc_epilogue_mm(accp, hs, degp, b, w):
  def body(accp_ref, hs_ref, degp_ref, b_ref, w_ref, o_ref):
    dinv = _dinv_from_deg(degp_ref)
    t = dinv * (accp_ref[0] + accp_ref[1] + hs_ref[...]) + b_ref[...]
    t = jnp.maximum(t, 0.0)
    o_ref[...] = jnp.dot(t, w_ref[...],
                         preferred_element_type=jnp.float32) * dinv

  return pl.pallas_call(
      body,
      grid=(_GRID,),
      in_specs=[
          pl.BlockSpec((NUM_CORES, _BLK, D), lambda i: (0, i, 0)),
          pl.BlockSpec((_BLK, D), lambda i: (i, 0)),
          pl.BlockSpec((NUM_CORES, _BLK, DEG_W), lambda i: (0, i, 0)),
          pl.BlockSpec((1, D), lambda i: (0, 0)),
          pl.BlockSpec((D, D), lambda i: (0, 0)),
      ],
      out_specs=pl.BlockSpec((_BLK, D), lambda i: (i, 0)),
      out_shape=jax.ShapeDtypeStruct((N, D), jnp.float32),
  )(accp, hs, degp, b, w)


def _tc_epilogue(accp, hs, degp, b):
  def body(accp_ref, hs_ref, degp_ref, b_ref, o_ref):
    dinv = _dinv_from_deg(degp_ref)
    t = dinv * (accp_ref[0] + accp_ref[1] + hs_ref[...]) + b_ref[...]
    o_ref[...] = jnp.maximum(t, 0.0)

  return pl.pallas_call(
      body,
      grid=(_GRID,),
      in_specs=[
          pl.BlockSpec((NUM_CORES, _BLK, D), lambda i: (0, i, 0)),
          pl.BlockSpec((_BLK, D), lambda i: (i, 0)),
          pl.BlockSpec((NUM_CORES, _BLK, DEG_W), lambda i: (0, i, 0)),
          pl.BlockSpec((1, D), lambda i: (0, 0)),
      ],
      out_specs=pl.BlockSpec((_BLK, D), lambda i: (i, 0)),
      out_shape=jax.ShapeDtypeStruct((N, D), jnp.float32),
  )(accp, hs, degp, b)


def kernel(x, edge_index, W1, b1, W2, b2):
  src = edge_index[0].astype(jnp.int32).reshape(IDX_ROWS, CHUNK)
  dst = edge_index[1].astype(jnp.int32).reshape(IDX_ROWS, CHUNK)
  b1r = b1.reshape(1, D)
  b2r = b2.reshape(1, D)

  degp = _sc_degree(dst)
  h1 = _tc_matmul(x, W1)

  hs1 = _tc_scale(h1, degp)
  acc1 = _sc_aggregate(hs1, src, dst)
  hs2 = _tc_epilogue_mm(acc1, hs1, degp, b1r, W2)
  acc2 = _sc_aggregate(hs2, src, dst)
  return _tc_epilogue(acc2, hs2, degp, b2r)


# speedup vs baseline: 2.5215x; 1.2267x over previous
import functools

import jax
import jax.numpy as jnp
from jax import lax
from jax.experimental import pallas as pl
from jax.experimental.pallas import tpu as pltpu
from jax.experimental.pallas import tpu_sc as plsc

N = 10000
E = 320000
D = 128

NUM_CORES = 2
NUM_SUBCORES = 16
ROW_CHUNK = 80
NUM_ROW_CHUNKS = N // ROW_CHUNK
ROW_CHUNKS_PER_SUBCORE = -(-NUM_ROW_CHUNKS // NUM_SUBCORES)
EDGES_PER_CORE = E // NUM_CORES
EDGES_PER_WORKER = EDGES_PER_CORE // NUM_SUBCORES
CHUNK = 125
CHUNKS_PER_WORKER = EDGES_PER_WORKER // CHUNK
IDX_ROWS = E // CHUNK
IDX_ROWS_PER_CORE = IDX_ROWS // NUM_CORES

A_CHUNK = 125
A_CHUNKS = EDGES_PER_WORKER // A_CHUNK
A_PHASES = 2
A_PHASE_CHUNKS = A_CHUNKS // A_PHASES
NUM_WORKERS = NUM_CORES * NUM_SUBCORES

DEG_W = 128


def _sc_mesh():
  return plsc.VectorSubcoreMesh(core_axis_name="c", subcore_axis_name="s")


def _sc_degree(dst):
  @functools.partial(
      pl.kernel,
      mesh=_sc_mesh(),
      out_type=jax.ShapeDtypeStruct((NUM_CORES, N, DEG_W), jnp.float32),
      scratch_types=[
          pltpu.VMEM((CHUNKS_PER_WORKER, CHUNK), jnp.int32),
          pltpu.VMEM((CHUNK, DEG_W), jnp.float32),
          pltpu.VMEM((ROW_CHUNK, DEG_W), jnp.float32),
          pltpu.VMEM_SHARED((N, DEG_W), jnp.float32),
          pltpu.SemaphoreType.DMA,
      ],
  )
  def deg_kernel(dst_hbm, out_hbm, idx_d, ones_v, zbuf, acc, sem):
    cid = lax.axis_index("c")
    sid = lax.axis_index("s")

    irow = cid * IDX_ROWS_PER_CORE + sid * CHUNKS_PER_WORKER
    pltpu.sync_copy(dst_hbm.at[pl.ds(irow, CHUNKS_PER_WORKER)], idx_d)

    @pl.loop(0, CHUNK)
    def _(r):
      @pl.loop(0, DEG_W, step=16)
      def _(j):
        ones_v[r, pl.ds(j, 16)] = jnp.ones((16,), jnp.float32)

    @pl.loop(0, ROW_CHUNK)
    def _(r):
      @pl.loop(0, DEG_W, step=16)
      def _(j):
        zbuf[r, pl.ds(j, 16)] = jnp.zeros((16,), jnp.float32)

    @pl.loop(0, ROW_CHUNKS_PER_SUBCORE)
    def _(z):
      c = z * NUM_SUBCORES + sid

      @pl.when(c < NUM_ROW_CHUNKS)
      def _():
        pltpu.sync_copy(zbuf, acc.at[pl.ds(c * ROW_CHUNK, ROW_CHUNK)])

    plsc.subcore_barrier()

    @pl.loop(0, CHUNKS_PER_WORKER)
    def _(i):
      pltpu.sync_copy(ones_v, acc.at[idx_d.at[i]], add=True)

    plsc.subcore_barrier()

    @pl.loop(0, ROW_CHUNKS_PER_SUBCORE)
    def _(z):
      c = z * NUM_SUBCORES + sid

      @pl.when(c < NUM_ROW_CHUNKS)
      def _():
        pltpu.sync_copy(
            acc.at[pl.ds(c * ROW_CHUNK, ROW_CHUNK)],
            out_hbm.at[cid].at[pl.ds(c * ROW_CHUNK, ROW_CHUNK)],
        )

  return deg_kernel(dst)


def _sc_aggregate(rows_hbm, src, dst):
  @functools.partial(
      pl.kernel,
      mesh=_sc_mesh(),
      out_type=jax.ShapeDtypeStruct((NUM_CORES, N, D), jnp.float32),
      scratch_types=[
          pltpu.VMEM((A_PHASE_CHUNKS, A_CHUNK), jnp.int32),
          pltpu.VMEM((A_PHASE_CHUNKS, A_CHUNK), jnp.int32),
          pltpu.VMEM((A_CHUNK, D), jnp.float32),
          pltpu.VMEM((A_CHUNK, D), jnp.float32),
          pltpu.VMEM_SHARED((N, D), jnp.float32),
          pltpu.SemaphoreType.DMA,
          pltpu.SemaphoreType.DMA,
      ],
  )
  def agg_kernel(rows_ref, src_hbm, dst_hbm, out_hbm,
                 idx_s, idx_d, rows0, rows1, acc, gsem0, gsem1):
    cid = lax.axis_index("c")
    sid = lax.axis_index("s")
    wid = cid * NUM_SUBCORES + sid

    @pl.loop(0, ROW_CHUNK)
    def _(r):
      @pl.loop(0, D, step=16)
      def _(j):
        rows0[r, pl.ds(j, 16)] = jnp.zeros((16,), jnp.float32)

    @pl.loop(0, ROW_CHUNKS_PER_SUBCORE)
    def _(z):
      c = z * NUM_SUBCORES + sid

      @pl.when(c < NUM_ROW_CHUNKS)
      def _():
        pltpu.sync_copy(rows0.at[pl.ds(0, ROW_CHUNK)],
                        acc.at[pl.ds(c * ROW_CHUNK, ROW_CHUNK)])

    plsc.subcore_barrier()

    for h in range(A_PHASES):
      pltpu.sync_copy(
          src_hbm.at[wid].at[pl.ds(h * A_PHASE_CHUNKS, A_PHASE_CHUNKS)],
          idx_s)
      pltpu.sync_copy(
          dst_hbm.at[wid].at[pl.ds(h * A_PHASE_CHUNKS, A_PHASE_CHUNKS)],
          idx_d)
      pltpu.async_copy(rows_ref.at[idx_s.at[0]], rows0, gsem0)

      @pl.loop(0, A_PHASE_CHUNKS, step=2)
      def _(i):
        pltpu.make_async_copy(rows_ref.at[idx_s.at[i]], rows0, gsem0).wait()
        pltpu.async_copy(rows_ref.at[idx_s.at[i + 1]], rows1, gsem1)
        pltpu.sync_copy(rows0, acc.at[idx_d.at[i]], add=True)

        pltpu.make_async_copy(
            rows_ref.at[idx_s.at[i + 1]], rows1, gsem1).wait()

        @pl.when(i + 2 < A_PHASE_CHUNKS)
        def _():
          pltpu.async_copy(rows_ref.at[idx_s.at[i + 2]], rows0, gsem0)

        pltpu.sync_copy(rows1, acc.at[idx_d.at[i + 1]], add=True)

    plsc.subcore_barrier()

    @pl.loop(0, ROW_CHUNKS_PER_SUBCORE)
    def _(z):
      c = z * NUM_SUBCORES + sid

      @pl.when(c < NUM_ROW_CHUNKS)
      def _():
        pltpu.sync_copy(
            acc.at[pl.ds(c * ROW_CHUNK, ROW_CHUNK)],
            out_hbm.at[cid].at[pl.ds(c * ROW_CHUNK, ROW_CHUNK)],
        )

  return agg_kernel(rows_hbm, src, dst)


_BLK = 1000
_GRID = N // _BLK


def _tc_matmul(x, w):
  def body(x_ref, w_ref, o_ref):
    o_ref[...] = jnp.dot(x_ref[...], w_ref[...],
                         preferred_element_type=jnp.float32)

  return pl.pallas_call(
      body,
      grid=(_GRID,),
      in_specs=[
          pl.BlockSpec((_BLK, D), lambda i: (i, 0)),
          pl.BlockSpec((D, D), lambda i: (0, 0)),
      ],
      out_specs=pl.BlockSpec((_BLK, D), lambda i: (i, 0)),
      out_shape=jax.ShapeDtypeStruct((N, D), jnp.float32),
  )(x, w)


def _dinv_from_deg(degp_ref):
  deg = degp_ref[0, :, 0:1] + degp_ref[1, :, 0:1] + 1.0
  return lax.rsqrt(deg)


def _tc_scale(h, degp):
  def body(h_ref, degp_ref, o_ref):
    o_ref[...] = h_ref[...] * _dinv_from_deg(degp_ref)

  return pl.pallas_call(
      body,
      grid=(_GRID,),
      in_specs=[
          pl.BlockSpec((_BLK, D), lambda i: (i, 0)),
          pl.BlockSpec((NUM_CORES, _BLK, DEG_W), lambda i: (0, i, 0)),
      ],
      out_specs=pl.BlockSpec((_BLK, D), lambda i: (i, 0)),
      out_shape=jax.ShapeDtypeStruct((N, D), jnp.float32),
  )(h, degp)


def _tc_epilogue_mm(accp, hs, degp, b, w):
  def body(accp_ref, hs_ref, degp_ref, b_ref, w_ref, o_ref):
    dinv = _dinv_from_deg(degp_ref)
    t = dinv * (accp_ref[0] + accp_ref[1] + hs_ref[...]) + b_ref[...]
    t = jnp.maximum(t, 0.0)
    o_ref[...] = jnp.dot(t, w_ref[...],
                         preferred_element_type=jnp.float32) * dinv

  return pl.pallas_call(
      body,
      grid=(_GRID,),
      in_specs=[
          pl.BlockSpec((NUM_CORES, _BLK, D), lambda i: (0, i, 0)),
          pl.BlockSpec((_BLK, D), lambda i: (i, 0)),
          pl.BlockSpec((NUM_CORES, _BLK, DEG_W), lambda i: (0, i, 0)),
          pl.BlockSpec((1, D), lambda i: (0, 0)),
          pl.BlockSpec((D, D), lambda i: (0, 0)),
      ],
      out_specs=pl.BlockSpec((_BLK, D), lambda i: (i, 0)),
      out_shape=jax.ShapeDtypeStruct((N, D), jnp.float32),
  )(accp, hs, degp, b, w)


def _tc_epilogue(accp, hs, degp, b):
  def body(accp_ref, hs_ref, degp_ref, b_ref, o_ref):
    dinv = _dinv_from_deg(degp_ref)
    t = dinv * (accp_ref[0] + accp_ref[1] + hs_ref[...]) + b_ref[...]
    o_ref[...] = jnp.maximum(t, 0.0)

  return pl.pallas_call(
      body,
      grid=(_GRID,),
      in_specs=[
          pl.BlockSpec((NUM_CORES, _BLK, D), lambda i: (0, i, 0)),
          pl.BlockSpec((_BLK, D), lambda i: (i, 0)),
          pl.BlockSpec((NUM_CORES, _BLK, DEG_W), lambda i: (0, i, 0)),
          pl.BlockSpec((1, D), lambda i: (0, 0)),
      ],
      out_specs=pl.BlockSpec((_BLK, D), lambda i: (i, 0)),
      out_shape=jax.ShapeDtypeStruct((N, D), jnp.float32),
  )(accp, hs, degp, b)


def kernel(x, edge_index, W1, b1, W2, b2):
  src_i = edge_index[0].astype(jnp.int32)
  dst_i = edge_index[1].astype(jnp.int32)
  dst = dst_i.reshape(IDX_ROWS, CHUNK)
  src_a = src_i.reshape(NUM_WORKERS, A_CHUNKS, A_CHUNK)
  dst_a = dst_i.reshape(NUM_WORKERS, A_CHUNKS, A_CHUNK)
  b1r = b1.reshape(1, D)
  b2r = b2.reshape(1, D)

  degp = _sc_degree(dst)
  h1 = _tc_matmul(x, W1)

  hs1 = _tc_scale(h1, degp)
  acc1 = _sc_aggregate(hs1, src_a, dst_a)
  hs2 = _tc_epilogue_mm(acc1, hs1, degp, b1r, W2)
  acc2 = _sc_aggregate(hs2, src_a, dst_a)
  return _tc_epilogue(acc2, hs2, degp, b2r)


# speedup vs baseline: 2.7082x; 1.0741x over previous
import functools

import jax
import jax.numpy as jnp
from jax import lax
from jax.experimental import pallas as pl
from jax.experimental.pallas import tpu as pltpu
from jax.experimental.pallas import tpu_sc as plsc

N = 10000
E = 320000
D = 128

NUM_CORES = 2
NUM_SUBCORES = 16
ROW_CHUNK = 80
NUM_ROW_CHUNKS = N // ROW_CHUNK
ROW_CHUNKS_PER_SUBCORE = -(-NUM_ROW_CHUNKS // NUM_SUBCORES)
EDGES_PER_CORE = E // NUM_CORES
EDGES_PER_WORKER = EDGES_PER_CORE // NUM_SUBCORES
CHUNK = 125
CHUNKS_PER_WORKER = EDGES_PER_WORKER // CHUNK
IDX_ROWS = E // CHUNK
IDX_ROWS_PER_CORE = IDX_ROWS // NUM_CORES

A_CHUNK = 125
A_CHUNKS = EDGES_PER_WORKER // A_CHUNK
A_PHASES = 2
A_PHASE_CHUNKS = A_CHUNKS // A_PHASES
NUM_WORKERS = NUM_CORES * NUM_SUBCORES

DEG_W = 64


def _sc_mesh():
  return plsc.VectorSubcoreMesh(core_axis_name="c", subcore_axis_name="s")


def _sc_degree(dst):
  @functools.partial(
      pl.kernel,
      mesh=_sc_mesh(),
      out_type=jax.ShapeDtypeStruct((NUM_CORES, N, DEG_W), jnp.float32),
      scratch_types=[
          pltpu.VMEM((CHUNKS_PER_WORKER, CHUNK), jnp.int32),
          pltpu.VMEM((CHUNK, DEG_W), jnp.float32),
          pltpu.VMEM((ROW_CHUNK, DEG_W), jnp.float32),
          pltpu.VMEM_SHARED((N, DEG_W), jnp.float32),
          pltpu.SemaphoreType.DMA,
      ],
  )
  def deg_kernel(dst_hbm, out_hbm, idx_d, ones_v, zbuf, acc, sem):
    cid = lax.axis_index("c")
    sid = lax.axis_index("s")

    irow = cid * IDX_ROWS_PER_CORE + sid * CHUNKS_PER_WORKER
    pltpu.sync_copy(dst_hbm.at[pl.ds(irow, CHUNKS_PER_WORKER)], idx_d)

    @pl.loop(0, CHUNK)
    def _(r):
      @pl.loop(0, DEG_W, step=16)
      def _(j):
        ones_v[r, pl.ds(j, 16)] = jnp.ones((16,), jnp.float32)

    @pl.loop(0, ROW_CHUNK)
    def _(r):
      @pl.loop(0, DEG_W, step=16)
      def _(j):
        zbuf[r, pl.ds(j, 16)] = jnp.zeros((16,), jnp.float32)

    @pl.loop(0, ROW_CHUNKS_PER_SUBCORE)
    def _(z):
      c = z * NUM_SUBCORES + sid

      @pl.when(c < NUM_ROW_CHUNKS)
      def _():
        pltpu.sync_copy(zbuf, acc.at[pl.ds(c * ROW_CHUNK, ROW_CHUNK)])

    plsc.subcore_barrier()

    @pl.loop(0, CHUNKS_PER_WORKER)
    def _(i):
      pltpu.sync_copy(ones_v, acc.at[idx_d.at[i]], add=True)

    plsc.subcore_barrier()

    @pl.loop(0, ROW_CHUNKS_PER_SUBCORE)
    def _(z):
      c = z * NUM_SUBCORES + sid

      @pl.when(c < NUM_ROW_CHUNKS)
      def _():
        pltpu.sync_copy(
            acc.at[pl.ds(c * ROW_CHUNK, ROW_CHUNK)],
            out_hbm.at[cid].at[pl.ds(c * ROW_CHUNK, ROW_CHUNK)],
        )

  return deg_kernel(dst)


def _sc_aggregate(rows_hbm, src, dst):
  @functools.partial(
      pl.kernel,
      mesh=_sc_mesh(),
      out_type=jax.ShapeDtypeStruct((NUM_CORES, N, D), jnp.float32),
      scratch_types=[
          pltpu.VMEM((A_PHASE_CHUNKS, A_CHUNK), jnp.int32),
          pltpu.VMEM((A_PHASE_CHUNKS, A_CHUNK), jnp.int32),
          pltpu.VMEM((A_CHUNK, D), jnp.float32),
          pltpu.VMEM((A_CHUNK, D), jnp.float32),
          pltpu.VMEM_SHARED((N, D), jnp.float32),
          pltpu.SemaphoreType.DMA,
          pltpu.SemaphoreType.DMA,
      ],
  )
  def agg_kernel(rows_ref, src_hbm, dst_hbm, out_hbm,
                 idx_s, idx_d, rows0, rows1, acc, gsem0, gsem1):
    cid = lax.axis_index("c")
    sid = lax.axis_index("s")
    wid = cid * NUM_SUBCORES + sid

    @pl.loop(0, ROW_CHUNK)
    def _(r):
      @pl.loop(0, D, step=16)
      def _(j):
        rows0[r, pl.ds(j, 16)] = jnp.zeros((16,), jnp.float32)

    @pl.loop(0, ROW_CHUNKS_PER_SUBCORE)
    def _(z):
      c = z * NUM_SUBCORES + sid

      @pl.when(c < NUM_ROW_CHUNKS)
      def _():
        pltpu.sync_copy(rows0.at[pl.ds(0, ROW_CHUNK)],
                        acc.at[pl.ds(c * ROW_CHUNK, ROW_CHUNK)])

    plsc.subcore_barrier()

    for h in range(A_PHASES):
      pltpu.sync_copy(
          src_hbm.at[wid].at[pl.ds(h * A_PHASE_CHUNKS, A_PHASE_CHUNKS)],
          idx_s)
      pltpu.sync_copy(
          dst_hbm.at[wid].at[pl.ds(h * A_PHASE_CHUNKS, A_PHASE_CHUNKS)],
          idx_d)
      pltpu.async_copy(rows_ref.at[idx_s.at[0]], rows0, gsem0)

      @pl.loop(0, A_PHASE_CHUNKS, step=2)
      def _(i):
        pltpu.make_async_copy(rows_ref.at[idx_s.at[i]], rows0, gsem0).wait()
        pltpu.async_copy(rows_ref.at[idx_s.at[i + 1]], rows1, gsem1)
        pltpu.sync_copy(rows0, acc.at[idx_d.at[i]], add=True)

        pltpu.make_async_copy(
            rows_ref.at[idx_s.at[i + 1]], rows1, gsem1).wait()

        @pl.when(i + 2 < A_PHASE_CHUNKS)
        def _():
          pltpu.async_copy(rows_ref.at[idx_s.at[i + 2]], rows0, gsem0)

        pltpu.sync_copy(rows1, acc.at[idx_d.at[i + 1]], add=True)

    plsc.subcore_barrier()

    @pl.loop(0, ROW_CHUNKS_PER_SUBCORE)
    def _(z):
      c = z * NUM_SUBCORES + sid

      @pl.when(c < NUM_ROW_CHUNKS)
      def _():
        pltpu.sync_copy(
            acc.at[pl.ds(c * ROW_CHUNK, ROW_CHUNK)],
            out_hbm.at[cid].at[pl.ds(c * ROW_CHUNK, ROW_CHUNK)],
        )

  return agg_kernel(rows_hbm, src, dst)


_BLK = 1000
_GRID = N // _BLK


def _tc_matmul(x, w):
  def body(x_ref, w_ref, o_ref):
    o_ref[...] = jnp.dot(x_ref[...], w_ref[...],
                         preferred_element_type=jnp.float32)

  return pl.pallas_call(
      body,
      grid=(_GRID,),
      in_specs=[
          pl.BlockSpec((_BLK, D), lambda i: (i, 0)),
          pl.BlockSpec((D, D), lambda i: (0, 0)),
      ],
      out_specs=pl.BlockSpec((_BLK, D), lambda i: (i, 0)),
      out_shape=jax.ShapeDtypeStruct((N, D), jnp.float32),
  )(x, w)


def _dinv_from_deg(degp_ref):
  deg = degp_ref[0, :, 0:1] + degp_ref[1, :, 0:1] + 1.0
  return lax.rsqrt(deg)


def _tc_scale(h, degp):
  def body(h_ref, degp_ref, o_ref):
    o_ref[...] = h_ref[...] * _dinv_from_deg(degp_ref)

  return pl.pallas_call(
      body,
      grid=(_GRID,),
      in_specs=[
          pl.BlockSpec((_BLK, D), lambda i: (i, 0)),
          pl.BlockSpec((NUM_CORES, _BLK, DEG_W), lambda i: (0, i, 0)),
      ],
      out_specs=pl.BlockSpec((_BLK, D), lambda i: (i, 0)),
      out_shape=jax.ShapeDtypeStruct((N, D), jnp.float32),
  )(h, degp)


def _tc_epilogue_mm(accp, hs, degp, b, w):
  def body(accp_ref, hs_ref, degp_ref, b_ref, w_ref, o_ref):
    dinv = _dinv_from_deg(degp_ref)
    t = dinv * (accp_ref[0] + accp_ref[1] + hs_ref[...]) + b_ref[...]
    t = jnp.maximum(t, 0.0)
    o_ref[...] = jnp.dot(t, w_ref[...],
                         preferred_element_type=jnp.float32) * dinv

  return pl.pallas_call(
      body,
      grid=(_GRID,),
      in_specs=[
          pl.BlockSpec((NUM_CORES, _BLK, D), lambda i: (0, i, 0)),
          pl.BlockSpec((_BLK, D), lambda i: (i, 0)),
          pl.BlockSpec((NUM_CORES, _BLK, DEG_W), lambda i: (0, i, 0)),
          pl.BlockSpec((1, D), lambda i: (0, 0)),
          pl.BlockSpec((D, D), lambda i: (0, 0)),
      ],
      out_specs=pl.BlockSpec((_BLK, D), lambda i: (i, 0)),
      out_shape=jax.ShapeDtypeStruct((N, D), jnp.float32),
  )(accp, hs, degp, b, w)


def _tc_epilogue(accp, hs, degp, b):
  def body(accp_ref, hs_ref, degp_ref, b_ref, o_ref):
    dinv = _dinv_from_deg(degp_ref)
    t = dinv * (accp_ref[0] + accp_ref[1] + hs_ref[...]) + b_ref[...]
    o_ref[...] = jnp.maximum(t, 0.0)

  return pl.pallas_call(
      body,
      grid=(_GRID,),
      in_specs=[
          pl.BlockSpec((NUM_CORES, _BLK, D), lambda i: (0, i, 0)),
          pl.BlockSpec((_BLK, D), lambda i: (i, 0)),
          pl.BlockSpec((NUM_CORES, _BLK, DEG_W), lambda i: (0, i, 0)),
          pl.BlockSpec((1, D), lambda i: (0, 0)),
      ],
      out_specs=pl.BlockSpec((_BLK, D), lambda i: (i, 0)),
      out_shape=jax.ShapeDtypeStruct((N, D), jnp.float32),
  )(accp, hs, degp, b)


def kernel(x, edge_index, W1, b1, W2, b2):
  src_i = edge_index[0].astype(jnp.int32)
  dst_i = edge_index[1].astype(jnp.int32)
  dst = dst_i.reshape(IDX_ROWS, CHUNK)
  src_a = src_i.reshape(NUM_WORKERS, A_CHUNKS, A_CHUNK)
  dst_a = dst_i.reshape(NUM_WORKERS, A_CHUNKS, A_CHUNK)
  b1r = b1.reshape(1, D)
  b2r = b2.reshape(1, D)

  degp = _sc_degree(dst)
  h1 = _tc_matmul(x, W1)

  hs1 = _tc_scale(h1, degp)
  acc1 = _sc_aggregate(hs1, src_a, dst_a)
  hs2 = _tc_epilogue_mm(acc1, hs1, degp, b1r, W2)
  acc2 = _sc_aggregate(hs2, src_a, dst_a)
  return _tc_epilogue(acc2, hs2, degp, b2r)
